# baseline (speedup 1.0000x reference)
import functools

import jax
import jax.numpy as jnp
from jax import lax
from jax.experimental import pallas as pl
from jax.experimental.pallas import tpu as pltpu
from jax.experimental.pallas import tpu_sc as plsc

_W = 128
_CW = 128
_NC, _NS = 2, 16
_NW = _NC * _NS


def _take16(vec, idx):
    dnums = lax.GatherDimensionNumbers(
        offset_dims=(), collapsed_slice_dims=(0,), start_index_map=(0,))
    return lax.gather(vec, idx[:, None], dnums, (1,),
                      mode=lax.GatherScatterMode.PROMISE_IN_BOUNDS)


def _sc_transpose(total, d):
    n = total * d
    nchunk = d // _CW
    per_w = total // _W * nchunk // _NW

    mesh = plsc.VectorSubcoreMesh(core_axis_name="c", subcore_axis_name="s")

    @functools.partial(
        pl.kernel,
        out_type=jax.ShapeDtypeStruct((n // _W, _W), jnp.float32),
        mesh=mesh,
        compiler_params=pltpu.CompilerParams(needs_layout_passes=False),
        scratch_types=[
            pltpu.VMEM((_W, _CW + 1), jnp.float32),
            pltpu.VMEM((_W, _CW + 1), jnp.float32),
            pltpu.VMEM((_CW, _W), jnp.float32),
            pltpu.VMEM((_CW, _W), jnp.float32),
            pltpu.VMEM((1, _CW), jnp.int32),
            pltpu.VMEM((1, _CW), jnp.int32),
            pltpu.VMEM((16,), jnp.int32),
            pltpu.SemaphoreType.DMA,
            pltpu.SemaphoreType.DMA,
            pltpu.SemaphoreType.DMA,
            pltpu.SemaphoreType.DMA,
        ],
    )
    def sc_kernel(x_hbm, info_hbm, out_hbm, in_v0, in_v1, tr_v0, tr_v1,
                  idx_v0, idx_v1, cu_v, in_s0, in_s1, sc_s0, sc_s1):
        bufs = ((in_v0, tr_v0, idx_v0, in_s0, sc_s0),
                (in_v1, tr_v1, idx_v1, in_s1, sc_s1))
        wid = lax.axis_index("s") * _NC + lax.axis_index("c")
        base_unit = wid * per_w
        iota = lax.iota(jnp.int32, 16)
        rows = [iota + 16 * h for h in range(_W // 16)]
        pltpu.sync_copy(info_hbm.at[pl.ds(0, 16)], cu_v)
        cu = cu_v[...]
        cu_next = jnp.where(iota == 15, jnp.int32(total),
                            _take16(cu, (iota + 1) & 15))

        def in_copy(u, in_v, sem):
            r0 = (u // nchunk) * _W
            c0 = (u % nchunk) * _CW
            return pltpu.make_async_copy(
                x_hbm.at[pl.ds(r0, _W), pl.ds(c0, _CW)],
                in_v.at[:, pl.ds(0, _CW)], sem)

        def scat_copy(tr_v, idx_v, sem):
            return pltpu.make_async_copy(
                tr_v, out_hbm.at[idx_v.at[0]], sem)

        in_copy(base_unit, in_v0, in_s0).start()
        in_copy(base_unit + 1, in_v1, in_s1).start()

        def outer(ii, carry):
            for b in range(2):
                in_v, tr_v, idx_v, in_s, sc_s = bufs[b]
                i = ii * 2 + b
                u = base_unit + i
                in_copy(u, in_v, in_s).wait()

                @pl.when(ii > 0)
                def _():
                    scat_copy(tr_v, idx_v, sc_s).wait()

                r0 = (u // nchunk) * _W
                c0 = (u % nchunk) * _CW
                s = plsc.all_reduce_population_count(cu <= r0) - 1
                seg_base = _take16(cu, s)
                seg_end = _take16(cu_next, s)
                ldiv = (seg_end - seg_base) // _W
                base_off = (seg_base * (d // _W) + (r0 - seg_base) // _W
                            + c0 * ldiv)

                for k in range(_CW // 16):
                    idx_v[0, pl.ds(k * 16, 16)] = (
                        base_off + (k * 16 + iota) * ldiv)

                def col_body(c2, c3):
                    colf = jnp.full((16,), c2)
                    for h in range(_W // 16):
                        vals = plsc.load_gather(in_v, [rows[h], colf])
                        tr_v[c2, pl.ds(h * 16, 16)] = vals
                    return c3

                lax.fori_loop(0, _CW, col_body, 0, unroll=False)

                @pl.when(ii < per_w // 2 - 1)
                def _():
                    in_copy(u + 2, in_v, in_s).start()

                scat_copy(tr_v, idx_v, sc_s).start()
            return carry

        lax.fori_loop(0, per_w // 2, outer, 0, unroll=False)

        for b in range(2):
            in_v, tr_v, idx_v, in_s, sc_s = bufs[b]
            scat_copy(tr_v, idx_v, sc_s).wait()

    return sc_kernel


def kernel(x, info):
    total, d = x.shape
    out2d = _sc_transpose(total, d)(x, info)
    return jnp.reshape(out2d, (total * d,))


# speedup vs baseline: 1.8058x; 1.8058x over previous
import functools

import jax
import jax.numpy as jnp
from jax import lax
from jax.experimental import pallas as pl
from jax.experimental.pallas import tpu as pltpu
from jax.experimental.pallas import tpu_sc as plsc

_W = 128
_CW = 128
_NC, _NS = 2, 16
_NW = _NC * _NS
_NB = 4


def _take16(vec, idx):
    dnums = lax.GatherDimensionNumbers(
        offset_dims=(), collapsed_slice_dims=(0,), start_index_map=(0,))
    return lax.gather(vec, idx[:, None], dnums, (1,),
                      mode=lax.GatherScatterMode.PROMISE_IN_BOUNDS)


def _tc_transpose(total, d):
    nt = total // _W

    def body(x_ref, o_ref):
        o_ref[...] = x_ref[...].T

    return pl.pallas_call(
        body,
        grid=(nt,),
        in_specs=[pl.BlockSpec((_W, d), lambda i: (i, 0))],
        out_specs=pl.BlockSpec((d, _W), lambda i: (i, 0)),
        out_shape=jax.ShapeDtypeStruct((nt * d, _W), jnp.float32),
    )


def _sc_scatter(total, d):
    n = total * d
    nchunk = d // _CW
    per_w = total // _W * nchunk // _NW

    mesh = plsc.VectorSubcoreMesh(core_axis_name="c", subcore_axis_name="s")

    @functools.partial(
        pl.kernel,
        out_type=jax.ShapeDtypeStruct((n // _W, _W), jnp.float32),
        mesh=mesh,
        compiler_params=pltpu.CompilerParams(needs_layout_passes=False,
                                             use_tc_tiling_on_sc=False),
        scratch_types=(
            [pltpu.VMEM((_CW, _W), jnp.float32) for _ in range(_NB)]
            + [pltpu.VMEM((1, _CW), jnp.int32) for _ in range(_NB)]
            + [pltpu.VMEM((16,), jnp.int32)]
            + [pltpu.SemaphoreType.DMA for _ in range(2 * _NB)]
        ),
    )
    def sc_kernel(stage_hbm, info_hbm, out_hbm, *refs):
        in_vs = refs[0:_NB]
        idx_vs = refs[_NB:2 * _NB]
        cu_v = refs[2 * _NB]
        in_ss = refs[2 * _NB + 1:3 * _NB + 1]
        sc_ss = refs[3 * _NB + 1:4 * _NB + 1]

        wid = lax.axis_index("s") * _NC + lax.axis_index("c")
        base_unit = wid * per_w
        iota = lax.iota(jnp.int32, 16)
        pltpu.sync_copy(info_hbm.at[pl.ds(0, 16)], cu_v)
        cu = cu_v[...]
        cu_next = jnp.where(iota == 15, jnp.int32(total),
                            _take16(cu, (iota + 1) & 15))

        def in_copy(u, b):
            t = u // nchunk
            c0 = (u % nchunk) * _CW
            return pltpu.make_async_copy(
                stage_hbm.at[pl.ds(t * d + c0, _CW)], in_vs[b], in_ss[b])

        def scat_copy(b):
            return pltpu.make_async_copy(
                in_vs[b], out_hbm.at[idx_vs[b].at[0]], sc_ss[b])

        in_copy(base_unit, 0).start()
        in_copy(base_unit + 1, 1).start()

        def outer(ii, carry):
            for b in range(_NB):
                u = base_unit + ii * _NB + b
                in_copy(u, b).wait()

                t = u // nchunk
                r0 = t * _W
                c0 = (u % nchunk) * _CW
                s = plsc.all_reduce_population_count(cu <= r0) - 1
                seg_base = _take16(cu, s)
                seg_end = _take16(cu_next, s)
                ldiv = (seg_end - seg_base) // _W
                base_off = (seg_base * (d // _W) + (r0 - seg_base) // _W
                            + c0 * ldiv)
                for k in range(_CW // 16):
                    idx_vs[b][0, pl.ds(k * 16, 16)] = (
                        base_off + (k * 16 + iota) * ldiv)

                scat_copy(b).start()

                bj = (b + 2) % _NB
                j_ok = (ii * _NB + b + 2) < per_w
                if b >= 2:
                    @pl.when(j_ok)
                    def _():
                        scat_copy(bj).wait()
                        in_copy(u + 2, bj).start()
                else:
                    @pl.when(jnp.logical_and(ii > 0, j_ok))
                    def _():
                        scat_copy(bj).wait()
                        in_copy(u + 2, bj).start()

                    @pl.when(jnp.logical_and(ii == 0, j_ok))
                    def _():
                        in_copy(u + 2, bj).start()
            return carry

        lax.fori_loop(0, per_w // _NB, outer, 0, unroll=False)

        for b in range(_NB):
            scat_copy(b).wait()

    return sc_kernel


def kernel(x, info):
    total, d = x.shape
    stage = _tc_transpose(total, d)(x)
    out2d = _sc_scatter(total, d)(stage, info)
    return jnp.reshape(out2d, (total * d,))


# speedup vs baseline: 3.1002x; 1.7168x over previous
import functools

import jax
import jax.numpy as jnp
from jax import lax
from jax.experimental import pallas as pl
from jax.experimental.pallas import tpu as pltpu
from jax.experimental.pallas import tpu_sc as plsc

_W = 128
_CW = 128
_NC, _NS = 2, 16
_NW = _NC * _NS
_NB = 4


def _take16(vec, idx):
    dnums = lax.GatherDimensionNumbers(
        offset_dims=(), collapsed_slice_dims=(0,), start_index_map=(0,))
    return lax.gather(vec, idx[:, None], dnums, (1,),
                      mode=lax.GatherScatterMode.PROMISE_IN_BOUNDS)


def _tc_transpose(total, d):
    bt = 4
    nt = total // _W

    def body(x_ref, o_ref):
        x4 = x_ref[...].reshape(bt, _W, d)
        o_ref[...] = jnp.transpose(x4, (0, 2, 1)).reshape(bt * d, _W)

    return pl.pallas_call(
        body,
        grid=(nt // bt,),
        in_specs=[pl.BlockSpec((bt * _W, d), lambda i: (i, 0))],
        out_specs=pl.BlockSpec((bt * d, _W), lambda i: (i, 0)),
        out_shape=jax.ShapeDtypeStruct((nt * d, _W), jnp.float32),
    )


def _sc_scatter(total, d):
    n = total * d
    nchunk = d // _CW
    per_w = total // _W * nchunk // _NW

    mesh = plsc.VectorSubcoreMesh(core_axis_name="c", subcore_axis_name="s")

    @functools.partial(
        pl.kernel,
        out_type=jax.ShapeDtypeStruct((n // _W, _W), jnp.float32),
        mesh=mesh,
        compiler_params=pltpu.CompilerParams(needs_layout_passes=False,
                                             use_tc_tiling_on_sc=False),
        scratch_types=(
            [pltpu.VMEM((_CW, _W), jnp.float32) for _ in range(_NB)]
            + [pltpu.VMEM((1, _CW), jnp.int32) for _ in range(_NB)]
            + [pltpu.VMEM((16,), jnp.int32)]
            + [pltpu.SemaphoreType.DMA for _ in range(2 * _NB)]
        ),
    )
    def sc_kernel(stage_hbm, info_hbm, out_hbm, *refs):
        in_vs = refs[0:_NB]
        idx_vs = refs[_NB:2 * _NB]
        cu_v = refs[2 * _NB]
        in_ss = refs[2 * _NB + 1:3 * _NB + 1]
        sc_ss = refs[3 * _NB + 1:4 * _NB + 1]

        wid = lax.axis_index("s") * _NC + lax.axis_index("c")
        base_unit = wid * per_w
        iota = lax.iota(jnp.int32, 16)
        pltpu.sync_copy(info_hbm.at[pl.ds(0, 16)], cu_v)
        cu = cu_v[...]
        cu_next = jnp.where(iota == 15, jnp.int32(total),
                            _take16(cu, (iota + 1) & 15))

        def in_copy(u, b):
            t = u // nchunk
            c0 = (u % nchunk) * _CW
            return pltpu.make_async_copy(
                stage_hbm.at[pl.ds(t * d + c0, _CW)], in_vs[b], in_ss[b])

        def scat_copy(b):
            return pltpu.make_async_copy(
                in_vs[b], out_hbm.at[idx_vs[b].at[0]], sc_ss[b])

        in_copy(base_unit, 0).start()
        in_copy(base_unit + 1, 1).start()

        def outer(ii, carry):
            for b in range(_NB):
                u = base_unit + ii * _NB + b
                in_copy(u, b).wait()

                t = u // nchunk
                r0 = t * _W
                c0 = (u % nchunk) * _CW
                s = plsc.all_reduce_population_count(cu <= r0) - 1
                seg_base = _take16(cu, s)
                seg_end = _take16(cu_next, s)
                ldiv = (seg_end - seg_base) // _W
                base_off = (seg_base * (d // _W) + (r0 - seg_base) // _W
                            + c0 * ldiv)
                for k in range(_CW // 16):
                    idx_vs[b][0, pl.ds(k * 16, 16)] = (
                        base_off + (k * 16 + iota) * ldiv)

                scat_copy(b).start()

                bj = (b + 2) % _NB
                j_ok = (ii * _NB + b + 2) < per_w
                if b >= 2:
                    @pl.when(j_ok)
                    def _():
                        scat_copy(bj).wait()
                        in_copy(u + 2, bj).start()
                else:
                    @pl.when(jnp.logical_and(ii > 0, j_ok))
                    def _():
                        scat_copy(bj).wait()
                        in_copy(u + 2, bj).start()

                    @pl.when(jnp.logical_and(ii == 0, j_ok))
                    def _():
                        in_copy(u + 2, bj).start()
            return carry

        lax.fori_loop(0, per_w // _NB, outer, 0, unroll=False)

        for b in range(_NB):
            scat_copy(b).wait()

    return sc_kernel


def kernel(x, info):
    total, d = x.shape
    stage = _tc_transpose(total, d)(x)
    out2d = _sc_scatter(total, d)(stage, info)
    return jnp.reshape(out2d, (total * d,))


# speedup vs baseline: 3.5929x; 1.1590x over previous
import functools

import jax
import jax.numpy as jnp
from jax import lax
from jax.experimental import pallas as pl
from jax.experimental.pallas import tpu as pltpu
from jax.experimental.pallas import tpu_sc as plsc

_W = 128
_CW = 128
_NC, _NS = 2, 16
_NW = _NC * _NS
_NB = 4


def _take16(vec, idx):
    dnums = lax.GatherDimensionNumbers(
        offset_dims=(), collapsed_slice_dims=(0,), start_index_map=(0,))
    return lax.gather(vec, idx[:, None], dnums, (1,),
                      mode=lax.GatherScatterMode.PROMISE_IN_BOUNDS)


def _tc_transpose(total, d):
    bt = 8
    nt = total // _W

    def body(x_ref, o_ref):
        x4 = x_ref[...].reshape(bt, _W, d)
        o_ref[...] = jnp.transpose(x4, (0, 2, 1)).reshape(bt * d, _W)

    return pl.pallas_call(
        body,
        grid=(nt // bt,),
        in_specs=[pl.BlockSpec((bt * _W, d), lambda i: (i, 0))],
        out_specs=pl.BlockSpec((bt * d, _W), lambda i: (i, 0)),
        out_shape=jax.ShapeDtypeStruct((nt * d, _W), jnp.float32),
    )


def _sc_scatter(total, d):
    n = total * d
    nchunk = d // _CW
    per_w = total // _W * nchunk // _NW

    mesh = plsc.VectorSubcoreMesh(core_axis_name="c", subcore_axis_name="s")

    @functools.partial(
        pl.kernel,
        out_type=jax.ShapeDtypeStruct((n // _W, _W), jnp.float32),
        mesh=mesh,
        compiler_params=pltpu.CompilerParams(needs_layout_passes=False,
                                             use_tc_tiling_on_sc=False),
        scratch_types=(
            [pltpu.VMEM((_CW, _W), jnp.float32) for _ in range(_NB)]
            + [pltpu.VMEM((1, _CW), jnp.int32) for _ in range(_NB)]
            + [pltpu.VMEM((16,), jnp.int32)]
            + [pltpu.SemaphoreType.DMA for _ in range(2 * _NB)]
        ),
    )
    def sc_kernel(stage_hbm, info_hbm, out_hbm, *refs):
        in_vs = refs[0:_NB]
        idx_vs = refs[_NB:2 * _NB]
        cu_v = refs[2 * _NB]
        in_ss = refs[2 * _NB + 1:3 * _NB + 1]
        sc_ss = refs[3 * _NB + 1:4 * _NB + 1]

        wid = lax.axis_index("s") * _NC + lax.axis_index("c")
        base_unit = wid * per_w
        iota = lax.iota(jnp.int32, 16)
        pltpu.sync_copy(info_hbm.at[pl.ds(0, 16)], cu_v)
        cu = cu_v[...]
        cu_next = jnp.where(iota == 15, jnp.int32(total),
                            _take16(cu, (iota + 1) & 15))

        def in_copy(u, b):
            t = u // nchunk
            c0 = (u % nchunk) * _CW
            return pltpu.make_async_copy(
                stage_hbm.at[pl.ds(t * d + c0, _CW)], in_vs[b], in_ss[b])

        def scat_copy(b):
            return pltpu.make_async_copy(
                in_vs[b], out_hbm.at[idx_vs[b].at[0]], sc_ss[b])

        in_copy(base_unit, 0).start()
        in_copy(base_unit + 1, 1).start()

        def outer(ii, carry):
            for b in range(_NB):
                u = base_unit + ii * _NB + b
                in_copy(u, b).wait()

                t = u // nchunk
                r0 = t * _W
                c0 = (u % nchunk) * _CW
                s = plsc.all_reduce_population_count(cu <= r0) - 1
                seg_base = _take16(cu, s)
                seg_end = _take16(cu_next, s)
                ldiv = (seg_end - seg_base) // _W
                base_off = (seg_base * (d // _W) + (r0 - seg_base) // _W
                            + c0 * ldiv)
                for k in range(_CW // 16):
                    idx_vs[b][0, pl.ds(k * 16, 16)] = (
                        base_off + (k * 16 + iota) * ldiv)

                scat_copy(b).start()

                bj = (b + 2) % _NB
                j_ok = (ii * _NB + b + 2) < per_w
                if b >= 2:
                    @pl.when(j_ok)
                    def _():
                        scat_copy(bj).wait()
                        in_copy(u + 2, bj).start()
                else:
                    @pl.when(jnp.logical_and(ii > 0, j_ok))
                    def _():
                        scat_copy(bj).wait()
                        in_copy(u + 2, bj).start()

                    @pl.when(jnp.logical_and(ii == 0, j_ok))
                    def _():
                        in_copy(u + 2, bj).start()
            return carry

        lax.fori_loop(0, per_w // _NB, outer, 0, unroll=False)

        for b in range(_NB):
            scat_copy(b).wait()

    return sc_kernel


def kernel(x, info):
    total, d = x.shape
    stage = _tc_transpose(total, d)(x)
    out2d = _sc_scatter(total, d)(stage, info)
    return jnp.reshape(out2d, (total * d,))


# speedup vs baseline: 3.8283x; 1.0655x over previous
import functools

import jax
import jax.numpy as jnp
from jax import lax
from jax.experimental import pallas as pl
from jax.experimental.pallas import tpu as pltpu
from jax.experimental.pallas import tpu_sc as plsc

_W = 128
_CW = 128
_NC, _NS = 2, 16
_NW = _NC * _NS
_NB = 4


def _take16(vec, idx):
    dnums = lax.GatherDimensionNumbers(
        offset_dims=(), collapsed_slice_dims=(0,), start_index_map=(0,))
    return lax.gather(vec, idx[:, None], dnums, (1,),
                      mode=lax.GatherScatterMode.PROMISE_IN_BOUNDS)


def _tc_transpose(total, d):
    bt = 16
    nt = total // _W

    def body(x_ref, o_ref):
        x4 = x_ref[...].reshape(bt, _W, d)
        o_ref[...] = jnp.transpose(x4, (0, 2, 1)).reshape(bt * d, _W)

    return pl.pallas_call(
        body,
        grid=(nt // bt,),
        in_specs=[pl.BlockSpec((bt * _W, d), lambda i: (i, 0))],
        out_specs=pl.BlockSpec((bt * d, _W), lambda i: (i, 0)),
        out_shape=jax.ShapeDtypeStruct((nt * d, _W), jnp.float32),
    )


def _sc_scatter(total, d):
    n = total * d
    nchunk = d // _CW
    per_w = total // _W * nchunk // _NW

    mesh = plsc.VectorSubcoreMesh(core_axis_name="c", subcore_axis_name="s")

    @functools.partial(
        pl.kernel,
        out_type=jax.ShapeDtypeStruct((n // _W, _W), jnp.float32),
        mesh=mesh,
        compiler_params=pltpu.CompilerParams(needs_layout_passes=False,
                                             use_tc_tiling_on_sc=False),
        scratch_types=(
            [pltpu.VMEM((_CW, _W), jnp.float32) for _ in range(_NB)]
            + [pltpu.VMEM((1, _CW), jnp.int32) for _ in range(_NB)]
            + [pltpu.VMEM((16,), jnp.int32)]
            + [pltpu.SemaphoreType.DMA for _ in range(2 * _NB)]
        ),
    )
    def sc_kernel(stage_hbm, info_hbm, out_hbm, *refs):
        in_vs = refs[0:_NB]
        idx_vs = refs[_NB:2 * _NB]
        cu_v = refs[2 * _NB]
        in_ss = refs[2 * _NB + 1:3 * _NB + 1]
        sc_ss = refs[3 * _NB + 1:4 * _NB + 1]

        wid = lax.axis_index("s") * _NC + lax.axis_index("c")
        base_unit = wid * per_w
        iota = lax.iota(jnp.int32, 16)
        pltpu.sync_copy(info_hbm.at[pl.ds(0, 16)], cu_v)
        cu = cu_v[...]
        cu_next = jnp.where(iota == 15, jnp.int32(total),
                            _take16(cu, (iota + 1) & 15))

        def in_copy(u, b):
            t = u // nchunk
            c0 = (u % nchunk) * _CW
            return pltpu.make_async_copy(
                stage_hbm.at[pl.ds(t * d + c0, _CW)], in_vs[b], in_ss[b])

        def scat_copy(b):
            return pltpu.make_async_copy(
                in_vs[b], out_hbm.at[idx_vs[b].at[0]], sc_ss[b])

        in_copy(base_unit, 0).start()
        in_copy(base_unit + 1, 1).start()

        def outer(ii, carry):
            for b in range(_NB):
                u = base_unit + ii * _NB + b
                in_copy(u, b).wait()

                t = u // nchunk
                r0 = t * _W
                c0 = (u % nchunk) * _CW
                s = plsc.all_reduce_population_count(cu <= r0) - 1
                seg_base = _take16(cu, s)
                seg_end = _take16(cu_next, s)
                ldiv = (seg_end - seg_base) // _W
                base_off = (seg_base * (d // _W) + (r0 - seg_base) // _W
                            + c0 * ldiv)
                for k in range(_CW // 16):
                    idx_vs[b][0, pl.ds(k * 16, 16)] = (
                        base_off + (k * 16 + iota) * ldiv)

                scat_copy(b).start()

                bj = (b + 2) % _NB
                j_ok = (ii * _NB + b + 2) < per_w
                if b >= 2:
                    @pl.when(j_ok)
                    def _():
                        scat_copy(bj).wait()
                        in_copy(u + 2, bj).start()
                else:
                    @pl.when(jnp.logical_and(ii > 0, j_ok))
                    def _():
                        scat_copy(bj).wait()
                        in_copy(u + 2, bj).start()

                    @pl.when(jnp.logical_and(ii == 0, j_ok))
                    def _():
                        in_copy(u + 2, bj).start()
            return carry

        lax.fori_loop(0, per_w // _NB, outer, 0, unroll=False)

        for b in range(_NB):
            scat_copy(b).wait()

    return sc_kernel


def kernel(x, info):
    total, d = x.shape
    stage = _tc_transpose(total, d)(x)
    out2d = _sc_scatter(total, d)(stage, info)
    return jnp.reshape(out2d, (total * d,))


# speedup vs baseline: 3.8503x; 1.0057x over previous
import functools

import jax
import jax.numpy as jnp
from jax import lax
from jax.experimental import pallas as pl
from jax.experimental.pallas import tpu as pltpu
from jax.experimental.pallas import tpu_sc as plsc

_W = 128
_CW = 128
_NC, _NS = 2, 16
_NW = _NC * _NS
_NB = 4


def _take16(vec, idx):
    dnums = lax.GatherDimensionNumbers(
        offset_dims=(), collapsed_slice_dims=(0,), start_index_map=(0,))
    return lax.gather(vec, idx[:, None], dnums, (1,),
                      mode=lax.GatherScatterMode.PROMISE_IN_BOUNDS)


def _tc_transpose(total, d):
    bt = 32
    nt = total // _W

    def body(x_ref, o_ref):
        x4 = x_ref[...].reshape(bt, _W, d)
        o_ref[...] = jnp.transpose(x4, (0, 2, 1)).reshape(bt * d, _W)

    return pl.pallas_call(
        body,
        grid=(nt // bt,),
        in_specs=[pl.BlockSpec((bt * _W, d), lambda i: (i, 0))],
        out_specs=pl.BlockSpec((bt * d, _W), lambda i: (i, 0)),
        out_shape=jax.ShapeDtypeStruct((nt * d, _W), jnp.float32),
    )


def _sc_scatter(total, d):
    n = total * d
    nchunk = d // _CW
    per_w = total // _W * nchunk // _NW

    mesh = plsc.VectorSubcoreMesh(core_axis_name="c", subcore_axis_name="s")

    @functools.partial(
        pl.kernel,
        out_type=jax.ShapeDtypeStruct((n // _W, _W), jnp.float32),
        mesh=mesh,
        compiler_params=pltpu.CompilerParams(needs_layout_passes=False,
                                             use_tc_tiling_on_sc=False),
        scratch_types=(
            [pltpu.VMEM((_CW, _W), jnp.float32) for _ in range(_NB)]
            + [pltpu.VMEM((1, _CW), jnp.int32) for _ in range(_NB)]
            + [pltpu.VMEM((16,), jnp.int32)]
            + [pltpu.SemaphoreType.DMA for _ in range(2 * _NB)]
        ),
    )
    def sc_kernel(stage_hbm, info_hbm, out_hbm, *refs):
        in_vs = refs[0:_NB]
        idx_vs = refs[_NB:2 * _NB]
        cu_v = refs[2 * _NB]
        in_ss = refs[2 * _NB + 1:3 * _NB + 1]
        sc_ss = refs[3 * _NB + 1:4 * _NB + 1]

        wid = lax.axis_index("s") * _NC + lax.axis_index("c")
        base_unit = wid * per_w
        iota = lax.iota(jnp.int32, 16)
        pltpu.sync_copy(info_hbm.at[pl.ds(0, 16)], cu_v)
        cu = cu_v[...]
        cu_next = jnp.where(iota == 15, jnp.int32(total),
                            _take16(cu, (iota + 1) & 15))

        def in_copy(u, b):
            t = u // nchunk
            c0 = (u % nchunk) * _CW
            return pltpu.make_async_copy(
                stage_hbm.at[pl.ds(t * d + c0, _CW)], in_vs[b], in_ss[b])

        def scat_copy(b):
            return pltpu.make_async_copy(
                in_vs[b], out_hbm.at[idx_vs[b].at[0]], sc_ss[b])

        in_copy(base_unit, 0).start()
        in_copy(base_unit + 1, 1).start()

        def outer(ii, carry):
            for b in range(_NB):
                u = base_unit + ii * _NB + b
                in_copy(u, b).wait()

                t = u // nchunk
                r0 = t * _W
                c0 = (u % nchunk) * _CW
                s = plsc.all_reduce_population_count(cu <= r0) - 1
                seg_base = _take16(cu, s)
                seg_end = _take16(cu_next, s)
                ldiv = (seg_end - seg_base) // _W
                base_off = (seg_base * (d // _W) + (r0 - seg_base) // _W
                            + c0 * ldiv)
                for k in range(_CW // 16):
                    idx_vs[b][0, pl.ds(k * 16, 16)] = (
                        base_off + (k * 16 + iota) * ldiv)

                scat_copy(b).start()

                bj = (b + 2) % _NB
                j_ok = (ii * _NB + b + 2) < per_w
                if b >= 2:
                    @pl.when(j_ok)
                    def _():
                        scat_copy(bj).wait()
                        in_copy(u + 2, bj).start()
                else:
                    @pl.when(jnp.logical_and(ii > 0, j_ok))
                    def _():
                        scat_copy(bj).wait()
                        in_copy(u + 2, bj).start()

                    @pl.when(jnp.logical_and(ii == 0, j_ok))
                    def _():
                        in_copy(u + 2, bj).start()
            return carry

        lax.fori_loop(0, per_w // _NB, outer, 0, unroll=False)

        for b in range(_NB):
            scat_copy(b).wait()

    return sc_kernel


def kernel(x, info):
    total, d = x.shape
    stage = _tc_transpose(total, d)(x)
    out2d = _sc_scatter(total, d)(stage, info)
    return jnp.reshape(out2d, (total * d,))
